# baseline (speedup 1.0000x reference)
import jax
import jax.numpy as jnp
from jax.experimental import pallas as pl
from jax.experimental.pallas import tpu as pltpu

_N = 16
_C = 255
_W = 128


def _body(a_ref, blo_ref, bhi_ref, c_ref, o_ref):
    k = pl.program_id(0)

    def emit(x):
        u = jnp.transpose(x, (1, 0, 2))
        v = jnp.transpose(u, (0, 2, 1))
        t = jnp.transpose(v, (1, 0, 2))
        o_ref[...] = 0.5 * jnp.tanh(0.5 * t) + 0.5
        for ch in (2, 3, 87, 88, 172, 173):
            o_ref[ch] = t[ch]

    @pl.when(k <= 44)
    def _():
        emit(a_ref[...])

    @pl.when(k == 45)
    def _():
        emit(jnp.concatenate([a_ref[0:16], blo_ref[0:112]], axis=0))

    @pl.when((k >= 46) & (k <= 55))
    def _():
        emit(jnp.concatenate([blo_ref[112:128], bhi_ref[0:112]], axis=0))

    @pl.when(k == 56)
    def _():
        emit(jnp.concatenate(
            [blo_ref[112:128], bhi_ref[0:36], c_ref[0:76]], axis=0))

    @pl.when(k == 57)
    def _():
        emit(c_ref[76:204])

    @pl.when(k == 58)
    def _():
        emit(c_ref[204:332])

    @pl.when(k == 59)
    def _():
        emit(jnp.concatenate([c_ref[332:361], c_ref[0:99]], axis=0))


def kernel(fm0, fm1, fm2, cell_anchors):
    del cell_anchors
    at = jnp.transpose(fm0, (2, 3, 0, 1)).reshape(5776, _N, _C)
    bt = jnp.transpose(fm1, (2, 3, 0, 1)).reshape(1444, _N, _C)
    ct = jnp.transpose(fm2, (2, 3, 0, 1)).reshape(361, _N, _C)
    out = pl.pallas_call(
        _body,
        grid=(60,),
        in_specs=[
            pl.BlockSpec((_W, _N, _C), lambda k: (jnp.minimum(k, 45), 0, 0)),
            pl.BlockSpec((_W, _N, _C), lambda k: (jnp.clip(k - 46, 0, 10), 0, 0)),
            pl.BlockSpec((_W, _N, _C), lambda k: (jnp.clip(k - 45, 0, 11), 0, 0)),
            pl.BlockSpec((361, _N, _C), lambda k: (0, 0, 0)),
        ],
        out_specs=pl.BlockSpec((_C, _N, _W), lambda k: (0, 0, k)),
        out_shape=jax.ShapeDtypeStruct((_C, _N, 7581), jnp.float32),
    )(at, bt, bt, ct)
    return jnp.transpose(out.reshape(3, 85, _N, 7581), (2, 3, 0, 1))


# speedup vs baseline: 1.1378x; 1.1378x over previous
import jax
import jax.numpy as jnp
from jax.experimental import pallas as pl
from jax.experimental.pallas import tpu as pltpu

_N = 16
_C = 255
_W = 256


def _act(x):
    c = jax.lax.broadcasted_iota(jnp.int32, x.shape, x.ndim - 1) % 85
    raw = (c == 2) | (c == 3)
    return jnp.where(raw, x, jax.nn.sigmoid(x))


def _body(a_ref, blo_ref, bhi_ref, c_ref, o_ref):
    k = pl.program_id(0)

    def emit(x):
        u = jnp.transpose(x, (1, 0, 2))
        v = jnp.transpose(u, (0, 2, 1))
        t = jnp.transpose(v, (1, 0, 2))
        o_ref[...] = 0.5 * jnp.tanh(0.5 * t) + 0.5
        for ch in (2, 3, 87, 88, 172, 173):
            o_ref[ch] = t[ch]

    @pl.when(k <= 21)
    def _():
        emit(a_ref[...])

    @pl.when(k == 22)
    def _():
        emit(jnp.concatenate([a_ref[0:144], blo_ref[0:112]], axis=0))

    @pl.when((k >= 23) & (k <= 27))
    def _():
        emit(jnp.concatenate([blo_ref[112:256], bhi_ref[0:112]], axis=0))

    @pl.when(k == 28)
    def _():
        emit(jnp.concatenate([blo_ref[112:164], c_ref[0:204]], axis=0))

    @pl.when(k == 29)
    def _():
        emit(jnp.concatenate([c_ref[204:361], c_ref[0:99]], axis=0))


def kernel(fm0, fm1, fm2, cell_anchors):
    del cell_anchors
    at = jnp.transpose(fm0, (2, 3, 0, 1)).reshape(5776, _N, _C)
    bt = jnp.transpose(fm1, (2, 3, 0, 1)).reshape(1444, _N, _C)
    ct = jnp.transpose(fm2, (2, 3, 0, 1)).reshape(361, _N, _C)
    out = pl.pallas_call(
        _body,
        grid=(30,),
        in_specs=[
            pl.BlockSpec((_W, _N, _C), lambda k: (jnp.minimum(k, 22), 0, 0)),
            pl.BlockSpec((_W, _N, _C), lambda k: (jnp.clip(k - 23, 0, 5), 0, 0)),
            pl.BlockSpec((_W, _N, _C), lambda k: (jnp.clip(k - 22, 0, 5), 0, 0)),
            pl.BlockSpec((361, _N, _C), lambda k: (0, 0, 0)),
        ],
        out_specs=pl.BlockSpec((_C, _N, _W), lambda k: (0, 0, k)),
        out_shape=jax.ShapeDtypeStruct((_C, _N, 7581), jnp.float32),
    )(at, bt, bt, ct)
    return jnp.transpose(out.reshape(3, 85, _N, 7581), (2, 3, 0, 1))


# speedup vs baseline: 1.1935x; 1.0489x over previous
import jax
import jax.numpy as jnp
from jax.experimental import pallas as pl
from jax.experimental.pallas import tpu as pltpu

_N = 16
_C = 255
_W = 256


def _body(a_ref, b_ref, c_ref, o_ref, tail_ref):
    k = pl.program_id(0)

    def emit(x):
        u = jnp.transpose(x, (1, 0, 2))
        v = jnp.transpose(u, (0, 2, 1))
        t = jnp.transpose(v, (1, 0, 2))
        o_ref[...] = 0.5 * jnp.tanh(0.5 * t) + 0.5
        for ch in (2, 3, 87, 88, 172, 173):
            o_ref[ch] = t[ch]

    @pl.when(k <= 21)
    def _():
        emit(a_ref[...])

    @pl.when(k == 22)
    def _():
        emit(jnp.concatenate([a_ref[0:144], b_ref[0:112]], axis=0))
        tail_ref[...] = b_ref[112:256]

    @pl.when((k >= 23) & (k <= 27))
    def _():
        emit(jnp.concatenate([tail_ref[...], b_ref[0:112]], axis=0))
        tail_ref[...] = b_ref[112:256]

    @pl.when(k == 28)
    def _():
        emit(jnp.concatenate([tail_ref[0:52], c_ref[0:204]], axis=0))

    @pl.when(k == 29)
    def _():
        emit(jnp.concatenate([c_ref[204:361], c_ref[0:99]], axis=0))


def kernel(fm0, fm1, fm2, cell_anchors):
    del cell_anchors
    at = jnp.transpose(fm0, (2, 3, 0, 1)).reshape(5776, _N, _C)
    bt = jnp.transpose(fm1, (2, 3, 0, 1)).reshape(1444, _N, _C)
    ct = jnp.transpose(fm2, (2, 3, 0, 1)).reshape(361, _N, _C)
    out = pl.pallas_call(
        _body,
        grid=(30,),
        in_specs=[
            pl.BlockSpec((_W, _N, _C), lambda k: (jnp.minimum(k, 22), 0, 0)),
            pl.BlockSpec((_W, _N, _C), lambda k: (jnp.clip(k - 22, 0, 5), 0, 0)),
            pl.BlockSpec((361, _N, _C), lambda k: (0, 0, 0)),
        ],
        out_specs=pl.BlockSpec((_C, _N, _W), lambda k: (0, 0, k)),
        out_shape=jax.ShapeDtypeStruct((_C, _N, 7581), jnp.float32),
        scratch_shapes=[pltpu.VMEM((144, _N, _C), jnp.float32)],
    )(at, bt, ct)
    return jnp.transpose(out.reshape(3, 85, _N, 7581), (2, 3, 0, 1))
